# speedup vs baseline: 1.6815x; 1.6815x over previous
import jax
import jax.numpy as jnp
from jax.experimental import pallas as pl

SEQ_LEN = 8192
EMBED_DIM = 768
BATCH = 4

S_BLK = 1024


def _add_kernel(x_ref, pos_ref, o_ref):
    o_ref[0] = x_ref[0] + pos_ref[...]


def kernel(inputs, pos_table):
    n_s = SEQ_LEN // S_BLK
    return pl.pallas_call(
        _add_kernel,
        grid=(n_s, BATCH),
        in_specs=[
            pl.BlockSpec((1, S_BLK, EMBED_DIM), lambda s, b: (b, s, 0)),
            pl.BlockSpec((S_BLK, EMBED_DIM), lambda s, b: (s, 0)),
        ],
        out_specs=pl.BlockSpec((1, S_BLK, EMBED_DIM), lambda s, b: (b, s, 0)),
        out_shape=jax.ShapeDtypeStruct((BATCH, SEQ_LEN, EMBED_DIM), jnp.float32),
    )(inputs, pos_table)


# speedup vs baseline: 1.7971x; 1.0687x over previous
import jax
import jax.numpy as jnp
from jax.experimental import pallas as pl

SEQ_LEN = 8192
EMBED_DIM = 768
BATCH = 4

S_BLK = 2048


def _add_kernel(x_ref, pos_ref, o_ref):
    o_ref[0] = x_ref[0] + pos_ref[...]


def kernel(inputs, pos_table):
    n_s = SEQ_LEN // S_BLK
    return pl.pallas_call(
        _add_kernel,
        grid=(n_s, BATCH),
        in_specs=[
            pl.BlockSpec((1, S_BLK, EMBED_DIM), lambda s, b: (b, s, 0)),
            pl.BlockSpec((S_BLK, EMBED_DIM), lambda s, b: (s, 0)),
        ],
        out_specs=pl.BlockSpec((1, S_BLK, EMBED_DIM), lambda s, b: (b, s, 0)),
        out_shape=jax.ShapeDtypeStruct((BATCH, SEQ_LEN, EMBED_DIM), jnp.float32),
    )(inputs, pos_table)


# speedup vs baseline: 1.8034x; 1.0035x over previous
import jax
import jax.numpy as jnp
from jax.experimental import pallas as pl

SEQ_LEN = 8192
EMBED_DIM = 768
BATCH = 4

S_BLK = 512


def _add_kernel(x_ref, pos_ref, o_ref):
    o_ref[...] = x_ref[...] + pos_ref[...][None]


def kernel(inputs, pos_table):
    n_s = SEQ_LEN // S_BLK
    return pl.pallas_call(
        _add_kernel,
        grid=(n_s,),
        in_specs=[
            pl.BlockSpec((BATCH, S_BLK, EMBED_DIM), lambda s: (0, s, 0)),
            pl.BlockSpec((S_BLK, EMBED_DIM), lambda s: (s, 0)),
        ],
        out_specs=pl.BlockSpec((BATCH, S_BLK, EMBED_DIM), lambda s: (0, s, 0)),
        out_shape=jax.ShapeDtypeStruct((BATCH, SEQ_LEN, EMBED_DIM), jnp.float32),
    )(inputs, pos_table)
